# baseline (speedup 1.0000x reference)
import functools
import jax
import jax.numpy as jnp
from jax import lax
from jax.experimental import pallas as pl
from jax.experimental.pallas import tpu as pltpu
from jax.experimental.pallas import tpu_sc as plsc

N = 10000
E = 320000
D = 128
NPAD = 10240
G = 64
NC = 2
NS = 16
EPW = E // (NC * NS)
EPS = E // NS
CHK = 80
F32 = jnp.float32


def _lk(v):
    return jnp.where(v >= 0, v, 0.2 * v)


def _nodeprep_body(x_ref, nt_ref, hw_ref, hb_ref, wij_ref, lw_ref,
                   sd_ref, m_ref):
    bn = x_ref.shape[0]
    xb = x_ref[...]
    nt = nt_ref[...]
    h = jnp.zeros((bn, D), F32)
    for t in range(8):
        ht = jnp.dot(xb, hw_ref[t], preferred_element_type=F32) + hb_ref[t][None, :]
        h = jnp.where(nt == t, ht, h)
    sd_ref[...] = jnp.dot(h, wij_ref[...], preferred_element_type=F32)
    m_ref[...] = jnp.dot(h, lw_ref[...], preferred_element_type=F32)


def _node_prep(xc, nt2d, hw, hb, wij, lw1):
    bn = 1000
    grid = N // bn
    return pl.pallas_call(
        _nodeprep_body,
        grid=(grid,),
        in_specs=[
            pl.BlockSpec((bn, D), lambda i: (i, 0)),
            pl.BlockSpec((bn, 1), lambda i: (i, 0)),
            pl.BlockSpec((8, D, D), lambda i: (0, 0, 0)),
            pl.BlockSpec((8, D), lambda i: (0, 0)),
            pl.BlockSpec((D, 2), lambda i: (0, 0)),
            pl.BlockSpec((D, D), lambda i: (0, 0)),
        ],
        out_specs=[
            pl.BlockSpec((bn, 2), lambda i: (i, 0)),
            pl.BlockSpec((bn, D), lambda i: (i, 0)),
        ],
        out_shape=[
            jax.ShapeDtypeStruct((N, 2), F32),
            jax.ShapeDtypeStruct((N, D), F32),
        ],
    )(xc, nt2d, hw, hb, wij, lw1)


def _edgeprep_body(ea_ref, et_ref, eaw_ref, ete_ref, wet_ref, wea_ref,
                   lw2_ref, me_ref, tl_ref):
    be = ea_ref.shape[0]
    ea2 = _lk(jnp.dot(ea_ref[...], eaw_ref[...], preferred_element_type=F32))
    me_ref[...] = jnp.dot(ea2, lw2_ref[...], preferred_element_type=F32)
    elt = jnp.dot(_lk(ete_ref[...]), wet_ref[...], preferred_element_type=F32)
    et = et_ref[...]
    oh = (et == lax.broadcasted_iota(jnp.int32, (be, 8), 1)).astype(F32)
    tl_ref[...] = (jnp.dot(ea2, wea_ref[...], preferred_element_type=F32)
                   + jnp.dot(oh, elt, preferred_element_type=F32))


def _edge_prep(ea, et2d, eaw, ete, wet, wea, lw2):
    be = 4000
    grid = E // be
    return pl.pallas_call(
        _edgeprep_body,
        grid=(grid,),
        in_specs=[
            pl.BlockSpec((be, 4), lambda i: (i, 0)),
            pl.BlockSpec((be, 1), lambda i: (i, 0)),
            pl.BlockSpec((4, 16), lambda i: (0, 0)),
            pl.BlockSpec((8, 16), lambda i: (0, 0)),
            pl.BlockSpec((16, 1), lambda i: (0, 0)),
            pl.BlockSpec((16, 1), lambda i: (0, 0)),
            pl.BlockSpec((16, D), lambda i: (0, 0)),
        ],
        out_specs=[
            pl.BlockSpec((be, D), lambda i: (i, 0)),
            pl.BlockSpec((be, 1), lambda i: (i, 0)),
        ],
        out_shape=[
            jax.ShapeDtypeStruct((E, D), F32),
            jax.ShapeDtypeStruct((E, 1), F32),
        ],
    )(ea, et2d, eaw, ete, wet, wea, lw2)


def _epi_body(op_ref, sig_ref, linb_ref, xn_ref):
    h = op_ref[...] + sig_ref[...] * linb_ref[...]
    xn_ref[...] = jnp.maximum(h, 0.0)


def _epilogue(opf, sig2d, linb2d):
    br = 1280
    grid = NPAD // br
    return pl.pallas_call(
        _epi_body,
        grid=(grid,),
        in_specs=[
            pl.BlockSpec((br, D), lambda i: (i, 0)),
            pl.BlockSpec((br, 1), lambda i: (i, 0)),
            pl.BlockSpec((1, D), lambda i: (0, 0)),
        ],
        out_specs=pl.BlockSpec((br, D), lambda i: (i, 0)),
        out_shape=jax.ShapeDtypeStruct((NPAD, D), F32),
    )(opf, sig2d, linb2d)


def _pool_body(x_ref, bt_ref, pw1_ref, pb1_ref, pw2_ref, pb2_ref, out_ref,
               acc, cnt):
    bn = x_ref.shape[0]
    i = pl.program_id(0)

    @pl.when(i == 0)
    def _():
        acc[...] = jnp.zeros_like(acc)
        cnt[...] = jnp.zeros_like(cnt)

    oh = (bt_ref[...] == lax.broadcasted_iota(jnp.int32, (bn, G), 1)).astype(F32)
    acc[...] += lax.dot_general(oh, x_ref[...], (((0,), (0,)), ((), ())),
                                preferred_element_type=F32)
    cnt[...] += lax.dot_general(oh, jnp.ones((bn, 1), F32),
                                (((0,), (0,)), ((), ())),
                                preferred_element_type=F32)

    @pl.when(i == pl.num_programs(0) - 1)
    def _():
        pooled = acc[...] / jnp.maximum(cnt[...], 1.0)
        t = jnp.dot(pooled, pw1_ref[...], preferred_element_type=F32) + pb1_ref[...]
        out_ref[...] = jnp.dot(t, pw2_ref[...], preferred_element_type=F32) + pb2_ref[...]


def _pool_mlp(xn, bt2d, pw1, pb1, pw2, pb2):
    bn = 1024
    grid = NPAD // bn
    return pl.pallas_call(
        _pool_body,
        grid=(grid,),
        in_specs=[
            pl.BlockSpec((bn, D), lambda i: (i, 0)),
            pl.BlockSpec((bn, 1), lambda i: (i, 0)),
            pl.BlockSpec((D, D), lambda i: (0, 0)),
            pl.BlockSpec((1, D), lambda i: (0, 0)),
            pl.BlockSpec((D, D), lambda i: (0, 0)),
            pl.BlockSpec((1, D), lambda i: (0, 0)),
        ],
        out_specs=pl.BlockSpec((G, D), lambda i: (0, 0)),
        out_shape=jax.ShapeDtypeStruct((G, D), F32),
        scratch_shapes=[pltpu.VMEM((G, D), F32), pltpu.VMEM((G, 1), F32)],
    )(xn, bt2d, pw1, pb1, pw2, pb2)


def _sc_scalar_body(dst_h, src_h, tl_h, sd_h, ss_h,
                    p_h, sp_h,
                    sd_t, ss_t, dst_t, src_t, tl_t, p_t, sloc,
                    tmp2d, acc_t, ssh):
    c = lax.axis_index("c")
    s = lax.axis_index("s")
    wid = s * NC + c
    base = wid * EPW

    pltpu.sync_copy(dst_h.at[pl.ds(base, EPW)], dst_t)
    pltpu.sync_copy(src_h.at[pl.ds(base, EPW)], src_t)
    pltpu.sync_copy(tl_h.at[pl.ds(base, EPW)], tl_t)
    pltpu.sync_copy(sd_h, sd_t)
    pltpu.sync_copy(ss_h, ss_t)

    zv = jnp.zeros((16,), F32)

    def zbody(i, _):
        sloc[pl.ds(i * 16, 16)] = zv
        return 0
    lax.fori_loop(0, NPAD // 16, zbody, 0)

    def ebody(i, _):
        sl = pl.ds(i * 16, 16)
        di = dst_t[sl]
        si = src_t[sl]
        a = (plsc.load_gather(sd_t, [di]) + plsc.load_gather(ss_t, [si])
             + tl_t[sl])
        pe = jnp.exp(_lk(a))
        p_t[sl] = pe
        plsc.addupdate_scatter(sloc, [di], pe)
        return 0
    lax.fori_loop(0, EPW // 16, ebody, 0)

    pltpu.sync_copy(p_t, p_h.at[pl.ds(base, EPW)])
    pltpu.sync_copy(sloc, ssh.at[s])
    plsc.subcore_barrier()
    cw = NPAD // NS
    pltpu.sync_copy(ssh.at[pl.ds(0, NS), pl.ds(s * cw, cw)], tmp2d)

    def mbody(j, _):
        sl = pl.ds(j * 16, 16)
        v = tmp2d[0, sl]
        for t in range(1, NS):
            v = v + tmp2d[t, sl]
        acc_t[sl] = v
        return 0
    lax.fori_loop(0, cw // 16, mbody, 0)
    pltpu.sync_copy(acc_t, sp_h.at[c, pl.ds(s * cw, cw)])


def _sc_scalar_phase(dst, src, tlog, sdst, ssrc):
    mesh = plsc.VectorSubcoreMesh(core_axis_name="c", subcore_axis_name="s")
    f = pl.kernel(
        _sc_scalar_body,
        mesh=mesh,
        compiler_params=pltpu.CompilerParams(needs_layout_passes=False),
        out_type=[
            jax.ShapeDtypeStruct((E,), F32),
            jax.ShapeDtypeStruct((NC, NPAD), F32),
        ],
        scratch_types=[
            pltpu.VMEM((N,), F32),
            pltpu.VMEM((N,), F32),
            pltpu.VMEM((EPW,), jnp.int32),
            pltpu.VMEM((EPW,), jnp.int32),
            pltpu.VMEM((EPW,), F32),
            pltpu.VMEM((EPW,), F32),
            pltpu.VMEM((NPAD,), F32),
            pltpu.VMEM((NS, NPAD // NS), F32),
            pltpu.VMEM((NPAD // NS,), F32),
            pltpu.VMEM_SHARED((NS, NPAD), F32),
        ],
    )
    return f(dst, src, tlog, sdst, ssrc)


HALF = NPAD // 2
HROWS = HALF + 8


def _sc_vector_body(dst_h, src_h, p_h, sp_h, m_h, me_h,
                    op_h, sig_h,
                    rs_t, sidx, didx, didx2, p_t, al_t, rows, me_t, zrow,
                    tmp32, srow, sgrow,
                    rsm, out_sh, lsem, gsem, ssem):
    c = lax.axis_index("c")
    s = lax.axis_index("s")

    cw = NPAD // NS
    pltpu.sync_copy(sp_h.at[pl.ds(0, NC), pl.ds(s * cw, cw)], tmp32)

    def mb(j, _):
        sl = pl.ds(j * 16, 16)
        v = tmp32[0, sl] + tmp32[1, sl]
        r = 1.0 / (v + 1e-16)
        srow[sl] = r
        sgrow[sl] = v * r
        return 0
    lax.fori_loop(0, cw // 16, mb, 0)
    pltpu.sync_copy(srow, rsm.at[pl.ds(s * cw, cw)])

    @pl.when(c == 0)
    def _():
        pltpu.sync_copy(sgrow, sig_h.at[pl.ds(s * cw, cw)])

    zv = jnp.zeros((16,), F32)

    def zr(i, _):
        for q in range(8):
            zrow[i, pl.ds(q * 16, 16)] = zv
        return 0
    lax.fori_loop(0, 160, zr, 0)

    rw = HALF // NS
    for q in range(2):
        pltpu.sync_copy(zrow, out_sh.at[pl.ds(s * rw + q * 160, 160)])

    @pl.when(s == 0)
    def _():
        pltpu.sync_copy(zrow.at[pl.ds(0, 8)], out_sh.at[pl.ds(HALF, 8)])
    plsc.subcore_barrier()
    pltpu.sync_copy(rsm, rs_t)

    lo = c * HALF
    nch = EPS // CHK

    def lin_pairs(k, b):
        base = s * EPS + k * CHK
        sl = pl.ds(base, CHK)
        return [(src_h.at[sl], sidx.at[b]), (dst_h.at[sl], didx.at[b]),
                (p_h.at[sl], p_t.at[b]), (me_h.at[sl], me_t.at[b])]

    def issue_lin(k, b):
        for sr, dr in lin_pairs(k, b):
            pltpu.make_async_copy(sr, dr, lsem.at[b]).start()

    def wait_lin(k, b):
        for sr, dr in lin_pairs(k, b):
            pltpu.make_async_copy(sr, dr, lsem.at[b]).wait()

    def gat(b):
        return pltpu.make_async_copy(m_h.at[sidx.at[b]], rows.at[b],
                                     gsem.at[b])

    def scat(b):
        return pltpu.make_async_copy(rows.at[b], out_sh.at[didx2.at[b]],
                                     ssem.at[b])

    def alphas(b):
        def ab(j, _):
            sl = pl.ds(j * 16, 16)
            dv = didx[b, sl]
            al_t[b, sl] = p_t[b, sl] * plsc.load_gather(rs_t, [dv])
            rel = dv - lo
            ok = (rel >= 0) & (rel < HALF)
            didx2[b, sl] = jnp.where(ok, rel, HALF)
            return 0
        lax.fori_loop(0, CHK // 16, ab, 0)

    def scale(b):
        def sc(j, _):
            av = al_t[b, pl.ds(j * 16, 16)]
            for l in range(16):
                row = j * 16 + l
                for q in range(8):
                    sl = pl.ds(q * 16, 16)
                    rows[b, row, sl] = ((rows[b, row, sl] + me_t[b, row, sl])
                                        * av[l])
            return 0
        lax.fori_loop(0, CHK // 16, sc, 0)

    issue_lin(0, 0)
    wait_lin(0, 0)
    gat(0).start()
    issue_lin(1, 1)

    def pair(g, _):
        for b in (0, 1):
            k = 2 * g + b
            b1 = 1 - b
            alphas(b)
            gat(b).wait()
            scale(b)

            @pl.when(2 * g + b + 2 < nch)
            def _():
                issue_lin(k + 2, b)

            @pl.when(2 * g + b + 1 < nch)
            def _():
                wait_lin(k + 1, b1)

            @pl.when(2 * g + b >= 1)
            def _():
                scat(b1).wait()

            @pl.when(2 * g + b + 1 < nch)
            def _():
                gat(b1).start()
            scat(b).start(add=True)
        return 0
    lax.fori_loop(0, nch // 2, pair, 0)
    scat((nch - 1) % 2).wait()

    plsc.subcore_barrier()
    pltpu.sync_copy(out_sh.at[pl.ds(s * rw, rw)], op_h.at[c, pl.ds(s * rw, rw)])


def _sc_vector_phase(dst, src, p, spart, m, me):
    mesh = plsc.VectorSubcoreMesh(core_axis_name="c", subcore_axis_name="s")
    f = pl.kernel(
        _sc_vector_body,
        mesh=mesh,
        compiler_params=pltpu.CompilerParams(needs_layout_passes=False),
        out_type=[
            jax.ShapeDtypeStruct((NC, HALF, D), F32),
            jax.ShapeDtypeStruct((NPAD,), F32),
        ],
        scratch_types=[
            pltpu.VMEM((NPAD,), F32),
            pltpu.VMEM((2, CHK), jnp.int32),
            pltpu.VMEM((2, CHK), jnp.int32),
            pltpu.VMEM((2, CHK), jnp.int32),
            pltpu.VMEM((2, CHK), F32),
            pltpu.VMEM((2, CHK), F32),
            pltpu.VMEM((2, CHK, D), F32),
            pltpu.VMEM((2, CHK, D), F32),
            pltpu.VMEM((160, D), F32),
            pltpu.VMEM((NC, NPAD // NS), F32),
            pltpu.VMEM((NPAD // NS,), F32),
            pltpu.VMEM((NPAD // NS,), F32),
            pltpu.VMEM_SHARED((NPAD,), F32),
            pltpu.VMEM_SHARED((HROWS, D), F32),
            pltpu.SemaphoreType.DMA((2,)),
            pltpu.SemaphoreType.DMA((2,)),
            pltpu.SemaphoreType.DMA((2,)),
        ],
    )
    return f(dst, src, p, spart, m, me)


def _layer(xc, src, dst, nt2d, me, tlog, hw, hb, attw, linw, linb):
    wij = attw[:2 * D].reshape(2, D).transpose()
    sdsrc, m = _node_prep(xc, nt2d, hw, hb, wij, linw[:D])
    sdst = sdsrc[:, 0] + 0.0
    ssrc = sdsrc[:, 1] + 0.0
    p, spart = _sc_scalar_phase(dst, src, tlog, sdst, ssrc)
    op, sig = _sc_vector_phase(dst, src, p, spart, m, me)
    xn = _epilogue(op.reshape(NPAD, D), sig.reshape(NPAD, 1),
                   linb.reshape(1, D))
    return xn


def kernel(x, edge_index, node_type, edge_type, edge_attr, batch,
           hw0, hb0, ete0, eaw0, attw0, linw0, linb0,
           hw1, hb1, ete1, eaw1, attw1, linw1, linb1,
           pw1, pb1, pw2, pb2):
    src = edge_index[0].astype(jnp.int32)
    dst = edge_index[1].astype(jnp.int32)
    nt2d = node_type.astype(jnp.int32).reshape(N, 1)
    et2d = edge_type.astype(jnp.int32).reshape(E, 1)

    me0, tl0 = _edge_prep(edge_attr, et2d, eaw0, ete0,
                          attw0[2 * D:2 * D + 16], attw0[2 * D + 16:],
                          linw0[D:])
    me1, tl1 = _edge_prep(edge_attr, et2d, eaw1, ete1,
                          attw1[2 * D:2 * D + 16], attw1[2 * D + 16:],
                          linw1[D:])

    h = _layer(x, src, dst, nt2d, me0, tl0.reshape(E),
               hw0, hb0, attw0, linw0, linb0)
    h = _layer(h[:N], src, dst, nt2d, me1, tl1.reshape(E),
               hw1, hb1, attw1, linw1, linb1)

    bt2d = jnp.concatenate([batch.astype(jnp.int32),
                            jnp.full((NPAD - N,), G, jnp.int32)]).reshape(NPAD, 1)
    return _pool_mlp(h, bt2d, pw1, pb1.reshape(1, D), pw2, pb2.reshape(1, D))


# speedup vs baseline: 1.2334x; 1.2334x over previous
import functools
import jax
import jax.numpy as jnp
from jax import lax
from jax.experimental import pallas as pl
from jax.experimental.pallas import tpu as pltpu
from jax.experimental.pallas import tpu_sc as plsc

N = 10000
E = 320000
D = 128
NPAD = 10240
G = 64
NC = 2
NS = 16
EPW = E // (NC * NS)
EPS = E // NS
CHK = 80
F32 = jnp.float32


def _lk(v):
    return jnp.where(v >= 0, v, 0.2 * v)


def _nodeprep_body(x_ref, nt_ref, hw_ref, hb_ref, wij_ref, lw_ref,
                   sd_ref, m_ref):
    bn = x_ref.shape[0]
    xb = x_ref[...]
    nt = nt_ref[...]
    h = jnp.zeros((bn, D), F32)
    for t in range(8):
        ht = jnp.dot(xb, hw_ref[t], preferred_element_type=F32) + hb_ref[t][None, :]
        h = jnp.where(nt == t, ht, h)
    sd_ref[...] = jnp.dot(h, wij_ref[...], preferred_element_type=F32)
    m_ref[...] = jnp.dot(h, lw_ref[...], preferred_element_type=F32)


def _node_prep(xc, nt2d, hw, hb, wij, lw1):
    bn = 1000
    grid = N // bn
    return pl.pallas_call(
        _nodeprep_body,
        grid=(grid,),
        in_specs=[
            pl.BlockSpec((bn, D), lambda i: (i, 0)),
            pl.BlockSpec((bn, 1), lambda i: (i, 0)),
            pl.BlockSpec((8, D, D), lambda i: (0, 0, 0)),
            pl.BlockSpec((8, D), lambda i: (0, 0)),
            pl.BlockSpec((D, 2), lambda i: (0, 0)),
            pl.BlockSpec((D, D), lambda i: (0, 0)),
        ],
        out_specs=[
            pl.BlockSpec((bn, 2), lambda i: (i, 0)),
            pl.BlockSpec((bn, D), lambda i: (i, 0)),
        ],
        out_shape=[
            jax.ShapeDtypeStruct((N, 2), F32),
            jax.ShapeDtypeStruct((N, D), F32),
        ],
    )(xc, nt2d, hw, hb, wij, lw1)


def _edgeprep_body(ea_ref, et_ref, eaw_ref, ete_ref, wet_ref, wea_ref,
                   lw2_ref, me_ref, tl_ref):
    be = ea_ref.shape[0]
    ea2 = _lk(jnp.dot(ea_ref[...], eaw_ref[...], preferred_element_type=F32))
    me_ref[...] = jnp.dot(ea2, lw2_ref[...], preferred_element_type=F32)
    elt = jnp.dot(_lk(ete_ref[...]), wet_ref[...], preferred_element_type=F32)
    et = et_ref[...]
    oh = (et == lax.broadcasted_iota(jnp.int32, (be, 8), 1)).astype(F32)
    tl_ref[...] = (jnp.dot(ea2, wea_ref[...], preferred_element_type=F32)
                   + jnp.dot(oh, elt, preferred_element_type=F32))


def _edge_prep(ea, et2d, eaw, ete, wet, wea, lw2):
    be = 4000
    grid = E // be
    return pl.pallas_call(
        _edgeprep_body,
        grid=(grid,),
        in_specs=[
            pl.BlockSpec((be, 4), lambda i: (i, 0)),
            pl.BlockSpec((be, 1), lambda i: (i, 0)),
            pl.BlockSpec((4, 16), lambda i: (0, 0)),
            pl.BlockSpec((8, 16), lambda i: (0, 0)),
            pl.BlockSpec((16, 1), lambda i: (0, 0)),
            pl.BlockSpec((16, 1), lambda i: (0, 0)),
            pl.BlockSpec((16, D), lambda i: (0, 0)),
        ],
        out_specs=[
            pl.BlockSpec((be, D), lambda i: (i, 0)),
            pl.BlockSpec((be, 1), lambda i: (i, 0)),
        ],
        out_shape=[
            jax.ShapeDtypeStruct((E, D), F32),
            jax.ShapeDtypeStruct((E, 1), F32),
        ],
    )(ea, et2d, eaw, ete, wet, wea, lw2)


def _epi_body(op_ref, sig_ref, linb_ref, xn_ref):
    h = op_ref[...] + sig_ref[...] * linb_ref[...]
    xn_ref[...] = jnp.maximum(h, 0.0)


def _epilogue(opf, sig2d, linb2d):
    br = 1280
    grid = NPAD // br
    return pl.pallas_call(
        _epi_body,
        grid=(grid,),
        in_specs=[
            pl.BlockSpec((br, D), lambda i: (i, 0)),
            pl.BlockSpec((br, 1), lambda i: (i, 0)),
            pl.BlockSpec((1, D), lambda i: (0, 0)),
        ],
        out_specs=pl.BlockSpec((br, D), lambda i: (i, 0)),
        out_shape=jax.ShapeDtypeStruct((NPAD, D), F32),
    )(opf, sig2d, linb2d)


def _pool_body(x_ref, bt_ref, pw1_ref, pb1_ref, pw2_ref, pb2_ref, out_ref,
               acc, cnt):
    bn = x_ref.shape[0]
    i = pl.program_id(0)

    @pl.when(i == 0)
    def _():
        acc[...] = jnp.zeros_like(acc)
        cnt[...] = jnp.zeros_like(cnt)

    oh = (bt_ref[...] == lax.broadcasted_iota(jnp.int32, (bn, G), 1)).astype(F32)
    acc[...] += lax.dot_general(oh, x_ref[...], (((0,), (0,)), ((), ())),
                                preferred_element_type=F32)
    cnt[...] += lax.dot_general(oh, jnp.ones((bn, 1), F32),
                                (((0,), (0,)), ((), ())),
                                preferred_element_type=F32)

    @pl.when(i == pl.num_programs(0) - 1)
    def _():
        pooled = acc[...] / jnp.maximum(cnt[...], 1.0)
        t = jnp.dot(pooled, pw1_ref[...], preferred_element_type=F32) + pb1_ref[...]
        out_ref[...] = jnp.dot(t, pw2_ref[...], preferred_element_type=F32) + pb2_ref[...]


def _pool_mlp(xn, bt2d, pw1, pb1, pw2, pb2):
    bn = 1024
    grid = NPAD // bn
    return pl.pallas_call(
        _pool_body,
        grid=(grid,),
        in_specs=[
            pl.BlockSpec((bn, D), lambda i: (i, 0)),
            pl.BlockSpec((bn, 1), lambda i: (i, 0)),
            pl.BlockSpec((D, D), lambda i: (0, 0)),
            pl.BlockSpec((1, D), lambda i: (0, 0)),
            pl.BlockSpec((D, D), lambda i: (0, 0)),
            pl.BlockSpec((1, D), lambda i: (0, 0)),
        ],
        out_specs=pl.BlockSpec((G, D), lambda i: (0, 0)),
        out_shape=jax.ShapeDtypeStruct((G, D), F32),
        scratch_shapes=[pltpu.VMEM((G, D), F32), pltpu.VMEM((G, 1), F32)],
    )(xn, bt2d, pw1, pb1, pw2, pb2)


def _sc_scalar_body(dst_h, src_h, tl_h, sd_h, ss_h,
                    p_h, sp_h,
                    sd_t, ss_t, dst_t, src_t, tl_t, p_t, sloc,
                    tmp2d, acc_t, ssh):
    c = lax.axis_index("c")
    s = lax.axis_index("s")
    wid = s * NC + c
    base = wid * EPW

    pltpu.sync_copy(dst_h.at[pl.ds(base, EPW)], dst_t)
    pltpu.sync_copy(src_h.at[pl.ds(base, EPW)], src_t)
    pltpu.sync_copy(tl_h.at[pl.ds(base, EPW)], tl_t)
    pltpu.sync_copy(sd_h, sd_t)
    pltpu.sync_copy(ss_h, ss_t)

    zv = jnp.zeros((16,), F32)

    def zbody(i, _):
        sloc[pl.ds(i * 16, 16)] = zv
        return 0
    lax.fori_loop(0, NPAD // 16, zbody, 0)

    def ebody(i, _):
        sl = pl.ds(i * 16, 16)
        di = dst_t[sl]
        si = src_t[sl]
        a = (plsc.load_gather(sd_t, [di]) + plsc.load_gather(ss_t, [si])
             + tl_t[sl])
        pe = jnp.exp(_lk(a))
        p_t[sl] = pe
        plsc.addupdate_scatter(sloc, [di], pe)
        return 0
    lax.fori_loop(0, EPW // 16, ebody, 0)

    pltpu.sync_copy(p_t, p_h.at[pl.ds(base, EPW)])
    pltpu.sync_copy(sloc, ssh.at[s])
    plsc.subcore_barrier()
    cw = NPAD // NS
    pltpu.sync_copy(ssh.at[pl.ds(0, NS), pl.ds(s * cw, cw)], tmp2d)

    def mbody(j, _):
        sl = pl.ds(j * 16, 16)
        v = tmp2d[0, sl]
        for t in range(1, NS):
            v = v + tmp2d[t, sl]
        acc_t[sl] = v
        return 0
    lax.fori_loop(0, cw // 16, mbody, 0)
    pltpu.sync_copy(acc_t, sp_h.at[c, pl.ds(s * cw, cw)])


def _sc_scalar_phase(dst, src, tlog, sdst, ssrc):
    mesh = plsc.VectorSubcoreMesh(core_axis_name="c", subcore_axis_name="s")
    f = pl.kernel(
        _sc_scalar_body,
        mesh=mesh,
        compiler_params=pltpu.CompilerParams(needs_layout_passes=False),
        out_type=[
            jax.ShapeDtypeStruct((E,), F32),
            jax.ShapeDtypeStruct((NC, NPAD), F32),
        ],
        scratch_types=[
            pltpu.VMEM((N,), F32),
            pltpu.VMEM((N,), F32),
            pltpu.VMEM((EPW,), jnp.int32),
            pltpu.VMEM((EPW,), jnp.int32),
            pltpu.VMEM((EPW,), F32),
            pltpu.VMEM((EPW,), F32),
            pltpu.VMEM((NPAD,), F32),
            pltpu.VMEM((NS, NPAD // NS), F32),
            pltpu.VMEM((NPAD // NS,), F32),
            pltpu.VMEM_SHARED((NS, NPAD), F32),
        ],
    )
    return f(dst, src, tlog, sdst, ssrc)


HALF = NPAD // 2
HROWS = HALF + 8


def _sc_vector_body(dst_h, src_h, p_h, sp_h, m_h, me_h,
                    op_h, sig_h,
                    rs_t, sidx, didx, didx2, p_t, al_t, rows, me_t, zrow,
                    tmp32, srow, sgrow,
                    rsm, out_sh, lsem, gsem, ssem):
    c = lax.axis_index("c")
    s = lax.axis_index("s")

    cw = NPAD // NS
    pltpu.sync_copy(sp_h.at[pl.ds(0, NC), pl.ds(s * cw, cw)], tmp32)

    def mb(j, _):
        sl = pl.ds(j * 16, 16)
        v = tmp32[0, sl] + tmp32[1, sl]
        r = 1.0 / (v + 1e-16)
        srow[sl] = r
        sgrow[sl] = v * r
        return 0
    lax.fori_loop(0, cw // 16, mb, 0)
    pltpu.sync_copy(srow, rsm.at[pl.ds(s * cw, cw)])

    @pl.when(c == 0)
    def _():
        pltpu.sync_copy(sgrow, sig_h.at[pl.ds(s * cw, cw)])

    zv = jnp.zeros((16,), F32)

    def zr(i, _):
        for q in range(8):
            zrow[i, pl.ds(q * 16, 16)] = zv
        return 0
    lax.fori_loop(0, 160, zr, 0)

    rw = HALF // NS
    for q in range(2):
        pltpu.sync_copy(zrow, out_sh.at[pl.ds(s * rw + q * 160, 160)])

    @pl.when(s == 0)
    def _():
        pltpu.sync_copy(zrow.at[pl.ds(0, 8)], out_sh.at[pl.ds(HALF, 8)])
    plsc.subcore_barrier()
    pltpu.sync_copy(rsm, rs_t)

    lo = c * HALF
    nch = EPS // CHK

    def lin_pairs(k, b):
        base = s * EPS + k * CHK
        sl = pl.ds(base, CHK)
        return [(src_h.at[sl], sidx.at[b]), (dst_h.at[sl], didx.at[b]),
                (p_h.at[sl], p_t.at[b]), (me_h.at[sl], me_t.at[b])]

    def issue_lin(k, b):
        for sr, dr in lin_pairs(k, b):
            pltpu.make_async_copy(sr, dr, lsem.at[b]).start()

    def wait_lin(k, b):
        for sr, dr in lin_pairs(k, b):
            pltpu.make_async_copy(sr, dr, lsem.at[b]).wait()

    def gat(b):
        return pltpu.make_async_copy(m_h.at[sidx.at[b]], rows.at[b],
                                     gsem.at[b])

    def scat(b):
        return pltpu.make_async_copy(rows.at[b], out_sh.at[didx2.at[b]],
                                     ssem.at[b])

    def alphas(b):
        def ab(j, _):
            sl = pl.ds(j * 16, 16)
            dv = didx[b, sl]
            al_t[b, sl] = p_t[b, sl] * plsc.load_gather(rs_t, [dv])
            rel = dv - lo
            ok = (rel >= 0) & (rel < HALF)
            didx2[b, sl] = jnp.where(ok, rel, HALF)
            return 0
        lax.fori_loop(0, CHK // 16, ab, 0)

    def scale(b):
        def sc(j, _):
            av = al_t[b, pl.ds(j * 16, 16)]
            dv2 = didx2[b, pl.ds(j * 16, 16)]
            for l in range(16):
                row = j * 16 + l

                def do_scale():
                    for q in range(8):
                        sl = pl.ds(q * 16, 16)
                        rows[b, row, sl] = ((rows[b, row, sl]
                                             + me_t[b, row, sl]) * av[l])
                lax.cond(dv2[l] != HALF, do_scale, lambda: None)
            return 0
        lax.fori_loop(0, CHK // 16, sc, 0)

    issue_lin(0, 0)
    wait_lin(0, 0)
    gat(0).start()
    issue_lin(1, 1)

    def pair(g, _):
        for b in (0, 1):
            k = 2 * g + b
            b1 = 1 - b
            alphas(b)
            gat(b).wait()
            scale(b)

            @pl.when(2 * g + b + 2 < nch)
            def _():
                issue_lin(k + 2, b)

            @pl.when(2 * g + b + 1 < nch)
            def _():
                wait_lin(k + 1, b1)

            @pl.when(2 * g + b >= 1)
            def _():
                scat(b1).wait()

            @pl.when(2 * g + b + 1 < nch)
            def _():
                gat(b1).start()
            scat(b).start(add=True)
        return 0
    lax.fori_loop(0, nch // 2, pair, 0)
    scat((nch - 1) % 2).wait()

    plsc.subcore_barrier()
    pltpu.sync_copy(out_sh.at[pl.ds(s * rw, rw)], op_h.at[c, pl.ds(s * rw, rw)])


def _sc_vector_phase(dst, src, p, spart, m, me):
    mesh = plsc.VectorSubcoreMesh(core_axis_name="c", subcore_axis_name="s")
    f = pl.kernel(
        _sc_vector_body,
        mesh=mesh,
        compiler_params=pltpu.CompilerParams(needs_layout_passes=False),
        out_type=[
            jax.ShapeDtypeStruct((NC, HALF, D), F32),
            jax.ShapeDtypeStruct((NPAD,), F32),
        ],
        scratch_types=[
            pltpu.VMEM((NPAD,), F32),
            pltpu.VMEM((2, CHK), jnp.int32),
            pltpu.VMEM((2, CHK), jnp.int32),
            pltpu.VMEM((2, CHK), jnp.int32),
            pltpu.VMEM((2, CHK), F32),
            pltpu.VMEM((2, CHK), F32),
            pltpu.VMEM((2, CHK, D), F32),
            pltpu.VMEM((2, CHK, D), F32),
            pltpu.VMEM((160, D), F32),
            pltpu.VMEM((NC, NPAD // NS), F32),
            pltpu.VMEM((NPAD // NS,), F32),
            pltpu.VMEM((NPAD // NS,), F32),
            pltpu.VMEM_SHARED((NPAD,), F32),
            pltpu.VMEM_SHARED((HROWS, D), F32),
            pltpu.SemaphoreType.DMA((2,)),
            pltpu.SemaphoreType.DMA((2,)),
            pltpu.SemaphoreType.DMA((2,)),
        ],
    )
    return f(dst, src, p, spart, m, me)


def _layer(xc, src, dst, nt2d, me, tlog, hw, hb, attw, linw, linb):
    wij = attw[:2 * D].reshape(2, D).transpose()
    sdsrc, m = _node_prep(xc, nt2d, hw, hb, wij, linw[:D])
    sdst = sdsrc[:, 0] + 0.0
    ssrc = sdsrc[:, 1] + 0.0
    p, spart = _sc_scalar_phase(dst, src, tlog, sdst, ssrc)
    op, sig = _sc_vector_phase(dst, src, p, spart, m, me)
    xn = _epilogue(op.reshape(NPAD, D), sig.reshape(NPAD, 1),
                   linb.reshape(1, D))
    return xn


def kernel(x, edge_index, node_type, edge_type, edge_attr, batch,
           hw0, hb0, ete0, eaw0, attw0, linw0, linb0,
           hw1, hb1, ete1, eaw1, attw1, linw1, linb1,
           pw1, pb1, pw2, pb2):
    src = edge_index[0].astype(jnp.int32)
    dst = edge_index[1].astype(jnp.int32)
    nt2d = node_type.astype(jnp.int32).reshape(N, 1)
    et2d = edge_type.astype(jnp.int32).reshape(E, 1)

    me0, tl0 = _edge_prep(edge_attr, et2d, eaw0, ete0,
                          attw0[2 * D:2 * D + 16], attw0[2 * D + 16:],
                          linw0[D:])
    me1, tl1 = _edge_prep(edge_attr, et2d, eaw1, ete1,
                          attw1[2 * D:2 * D + 16], attw1[2 * D + 16:],
                          linw1[D:])

    h = _layer(x, src, dst, nt2d, me0, tl0.reshape(E),
               hw0, hb0, attw0, linw0, linb0)
    h = _layer(h[:N], src, dst, nt2d, me1, tl1.reshape(E),
               hw1, hb1, attw1, linw1, linb1)

    bt2d = jnp.concatenate([batch.astype(jnp.int32),
                            jnp.full((NPAD - N,), G, jnp.int32)]).reshape(NPAD, 1)
    return _pool_mlp(h, bt2d, pw1, pb1.reshape(1, D), pw2, pb2.reshape(1, D))
